# baseline (speedup 1.0000x reference)
import functools

import jax
import jax.numpy as jnp
from jax import lax
from jax.experimental import pallas as pl
from jax.experimental.pallas import tpu as pltpu
from jax.experimental.pallas import tpu_sc as plsc

B, T = 1024, 200
NRG, NSG, NPC, EMB = 32, 32, 64, 128


TB = 8
GRID = T // TB


def _tc_body(r_ref, s_ref, p_ref, w_ref, emb_ref):
    tt = pl.program_id(0) % TB
    r = r_ref[tt][None, :]
    s = s_ref[tt][None, :]
    p = p_ref[tt][None, :]
    i128 = lax.broadcasted_iota(jnp.int32, (EMB, B), 0)
    tg = ((i128 == r) | (i128 == s + NRG) | (i128 == p + NRG + NSG)
          ).astype(jnp.bfloat16)
    w = w_ref[...].astype(jnp.bfloat16)
    emb_ref[0] = lax.dot_general(
        w, tg, (((1,), (0,)), ((), ())),
        preferred_element_type=jnp.float32)


def _tc_emb(rT, sT, pT, W):
    idx_spec = pl.BlockSpec((TB, B), lambda i: (i // TB, 0))
    return pl.pallas_call(
        _tc_body,
        grid=(T,),
        in_specs=[idx_spec, idx_spec, idx_spec,
                  pl.BlockSpec((EMB, EMB), lambda i: (0, 0))],
        out_specs=pl.BlockSpec((1, EMB, B), lambda i: (i, 0, 0)),
        out_shape=jax.ShapeDtypeStruct((T, EMB, B), jnp.float32),
    )(rT, sT, pT, W)


NC, NS = 2, 16
NW = NC * NS
TW = T // 8
KW = 32
NJ = B // 16

_sc_mesh = plsc.VectorSubcoreMesh(core_axis_name="c", subcore_axis_name="s")


@functools.partial(
    pl.kernel,
    out_type=(jax.ShapeDtypeStruct((T, NRG * B), jnp.float32),
              jax.ShapeDtypeStruct((T, NSG * B), jnp.float32),
              jax.ShapeDtypeStruct((T, NPC * B), jnp.float32)),
    mesh=_sc_mesh,
    compiler_params=pltpu.CompilerParams(needs_layout_passes=False),
    scratch_types=[pltpu.VMEM((B,), jnp.int32),
                   pltpu.VMEM((B,), jnp.int32),
                   pltpu.VMEM((B,), jnp.int32),
                   pltpu.VMEM((B,), jnp.int32),
                   pltpu.VMEM((KW * B + 16,), jnp.float32),
                   pltpu.VMEM((KW * B + 16,), jnp.float32),
                   pltpu.SemaphoreType.DMA,
                   pltpu.SemaphoreType.DMA,
                   pltpu.SemaphoreType.DMA,
                   pltpu.SemaphoreType.DMA],
)
def _sc_onehots(rT_hbm, sT_hbm, pT_hbm, r_out, s_out, p_out,
                ivA, ivB, oldA, oldB, slabA, slabB, semIA, semIB, semA, semB):
    wid = lax.axis_index("s") * NC + lax.axis_index("c")
    which = wid >> 3
    t0 = (wid & 7) * TW

    one = jnp.full((16,), 1.0, jnp.float32)
    zero = jnp.zeros((16,), jnp.float32)
    zero_i = jnp.zeros((16,), jnp.int32)
    i16 = lax.broadcasted_iota(jnp.int32, (16,), 0)

    def _zrow(q, c):
        slabA[pl.ds(q * 16, 16)] = zero
        slabB[pl.ds(q * 16, 16)] = zero
        return c
    lax.fori_loop(0, KW * NJ, _zrow, 0)

    def _zi(j, c):
        oldA[pl.ds(j * 16, 16)] = zero_i
        oldB[pl.ds(j * 16, 16)] = zero_i
        return c
    lax.fori_loop(0, NJ, _zi, 0)

    def run(src_hbm, dst_at, kbase, banded):
        pltpu.async_copy(src_hbm.at[t0], ivA, semIA)

        def build(slab, iv, old):
            def inj(j, c):
                cols = i16 + j * 16
                vo = old[pl.ds(j * 16, 16)]
                v = iv[pl.ds(j * 16, 16)]
                if banded:
                    ro = vo - kbase
                    rn = v - kbase
                    fo = jnp.where((ro >= 0) & (ro < KW), ro * B + cols,
                                   KW * B + i16)
                    fn = jnp.where((rn >= 0) & (rn < KW), rn * B + cols,
                                   KW * B + i16)
                    plsc.store_scatter(slab, [fo], zero)
                    plsc.store_scatter(slab, [fn], one)
                else:
                    plsc.store_scatter(slab, [vo * B + cols], zero)
                    plsc.store_scatter(slab, [v * B + cols], one)
                old[pl.ds(j * 16, 16)] = v
                return c

            lax.fori_loop(0, NJ, inj, 0)

        def step(i, c):
            t = t0 + i

            @pl.when((i & 1) == 0)
            def _():
                pltpu.make_async_copy(src_hbm.at[t0], ivA, semIA).wait()

                @pl.when(i + 1 < TW)
                def _():
                    pltpu.async_copy(src_hbm.at[t + 1], ivB, semIB)

                @pl.when(i >= 2)
                def _():
                    pltpu.make_async_copy(slabA.at[pl.ds(0, KW * B)], dst_at(t0), semA).wait()
                build(slabA, ivA, oldA)
                pltpu.async_copy(slabA.at[pl.ds(0, KW * B)], dst_at(t), semA)

            @pl.when((i & 1) == 1)
            def _():
                pltpu.make_async_copy(src_hbm.at[t0], ivB, semIB).wait()

                @pl.when(i + 1 < TW)
                def _():
                    pltpu.async_copy(src_hbm.at[t + 1], ivA, semIA)

                @pl.when(i >= 2)
                def _():
                    pltpu.make_async_copy(slabB.at[pl.ds(0, KW * B)], dst_at(t0), semB).wait()
                build(slabB, ivB, oldB)
                pltpu.async_copy(slabB.at[pl.ds(0, KW * B)], dst_at(t), semB)

            return c

        lax.fori_loop(0, TW, step, 0)
        pltpu.make_async_copy(slabA.at[pl.ds(0, KW * B)], dst_at(t0), semA).wait()
        pltpu.make_async_copy(slabB.at[pl.ds(0, KW * B)], dst_at(t0), semB).wait()

    lax.switch(which, [
        lambda: run(rT_hbm, lambda t: r_out.at[t], 0, False),
        lambda: run(sT_hbm, lambda t: s_out.at[t], 0, False),
        lambda: run(pT_hbm, lambda t: p_out.at[t, pl.ds(0, KW * B)], 0, True),
        lambda: run(pT_hbm, lambda t: p_out.at[t, pl.ds(KW * B, KW * B)],
                    KW, True),
    ])


def kernel(rgap, sgap, pcount, W):
    rT = rgap.T
    sT = sgap.T
    pT = pcount.T
    r_oh, s_oh, p_oh = _sc_onehots(rT, sT, pT)
    emb = _tc_emb(rT, sT, pT, W)
    r_oh = r_oh.reshape(T, NRG, B).transpose(2, 0, 1)
    s_oh = s_oh.reshape(T, NSG, B).transpose(2, 0, 1)
    p_oh = p_oh.reshape(T, NPC, B).transpose(2, 0, 1)
    return (r_oh, s_oh, p_oh, emb.transpose(2, 0, 1))


# speedup vs baseline: 1.2805x; 1.2805x over previous
import functools

import jax
import jax.numpy as jnp
from jax import lax
from jax.experimental import pallas as pl
from jax.experimental.pallas import tpu as pltpu
from jax.experimental.pallas import tpu_sc as plsc

B, T = 1024, 200
NRG, NSG, NPC, EMB = 32, 32, 64, 128


TB = 8
GRID = T // TB


def _tc_body(r_ref, s_ref, p_ref, w_ref, emb_ref, soh_ref, poh_ref):
    tt = pl.program_id(0) % TB
    r = r_ref[tt][None, :]
    s = s_ref[tt][None, :]
    p = p_ref[tt][None, :]
    i128 = lax.broadcasted_iota(jnp.int32, (EMB, B), 0)
    tg = ((i128 == r) | (i128 == s + NRG) | (i128 == p + NRG + NSG)
          ).astype(jnp.bfloat16)
    w = w_ref[...].astype(jnp.bfloat16)
    emb_ref[0] = lax.dot_general(
        w, tg, (((1,), (0,)), ((), ())),
        preferred_element_type=jnp.float32)
    oh = tg.astype(jnp.float32)
    soh_ref[0] = oh[NRG:NRG + NSG]
    poh_ref[0] = oh[NRG + NSG:]


def _tc_emb(rT, sT, pT, W):
    idx_spec = pl.BlockSpec((TB, B), lambda i: (i // TB, 0))
    return pl.pallas_call(
        _tc_body,
        grid=(T,),
        in_specs=[idx_spec, idx_spec, idx_spec,
                  pl.BlockSpec((EMB, EMB), lambda i: (0, 0))],
        out_specs=[pl.BlockSpec((1, EMB, B), lambda i: (i, 0, 0)),
                   pl.BlockSpec((1, NSG, B), lambda i: (i, 0, 0)),
                   pl.BlockSpec((1, NPC, B), lambda i: (i, 0, 0))],
        out_shape=[jax.ShapeDtypeStruct((T, EMB, B), jnp.float32),
                   jax.ShapeDtypeStruct((T, NSG, B), jnp.float32),
                   jax.ShapeDtypeStruct((T, NPC, B), jnp.float32)],
    )(rT, sT, pT, W)


NC, NS = 2, 16
NW = NC * NS
TW = T // 8
KW = 32
NJ = B // 16

_sc_mesh = plsc.VectorSubcoreMesh(core_axis_name="c", subcore_axis_name="s")


@functools.partial(
    pl.kernel,
    out_type=jax.ShapeDtypeStruct((T, NRG * B), jnp.float32),
    mesh=_sc_mesh,
    compiler_params=pltpu.CompilerParams(needs_layout_passes=False),
    scratch_types=[pltpu.VMEM((B,), jnp.int32),
                   pltpu.VMEM((B,), jnp.int32),
                   pltpu.VMEM((B,), jnp.int32),
                   pltpu.VMEM((B,), jnp.int32),
                   pltpu.VMEM((KW * B + 16,), jnp.float32),
                   pltpu.VMEM((KW * B + 16,), jnp.float32),
                   pltpu.SemaphoreType.DMA,
                   pltpu.SemaphoreType.DMA,
                   pltpu.SemaphoreType.DMA,
                   pltpu.SemaphoreType.DMA],
)
def _sc_onehots(rT_hbm, r_out,
                ivA, ivB, oldA, oldB, slabA, slabB, semIA, semIB, semA, semB):
    wid = lax.axis_index("s") * NC + lax.axis_index("c")
    w8 = wid < 8
    t0 = jnp.where(w8, 7 * wid, 8 + 6 * wid)
    cnt = jnp.where(w8, 7, 6)

    one = jnp.full((16,), 1.0, jnp.float32)
    zero = jnp.zeros((16,), jnp.float32)
    zero_i = jnp.zeros((16,), jnp.int32)
    i16 = lax.broadcasted_iota(jnp.int32, (16,), 0)

    def _zrow(q, c):
        slabA[pl.ds(q * 16, 16)] = zero
        slabB[pl.ds(q * 16, 16)] = zero
        return c
    lax.fori_loop(0, KW * NJ, _zrow, 0)

    def _zi(j, c):
        oldA[pl.ds(j * 16, 16)] = zero_i
        oldB[pl.ds(j * 16, 16)] = zero_i
        return c
    lax.fori_loop(0, NJ, _zi, 0)

    def run(src_hbm, dst_at, kbase, banded):
        pltpu.async_copy(src_hbm.at[t0], ivA, semIA)

        def build(slab, iv, old):
            def inj(j, c):
                cols = i16 + j * 16
                vo = old[pl.ds(j * 16, 16)]
                v = iv[pl.ds(j * 16, 16)]
                if banded:
                    ro = vo - kbase
                    rn = v - kbase
                    fo = jnp.where((ro >= 0) & (ro < KW), ro * B + cols,
                                   KW * B + i16)
                    fn = jnp.where((rn >= 0) & (rn < KW), rn * B + cols,
                                   KW * B + i16)
                    plsc.store_scatter(slab, [fo], zero)
                    plsc.store_scatter(slab, [fn], one)
                else:
                    plsc.store_scatter(slab, [vo * B + cols], zero)
                    plsc.store_scatter(slab, [v * B + cols], one)
                old[pl.ds(j * 16, 16)] = v
                return c

            lax.fori_loop(0, NJ, inj, 0)

        def step(i, c):
            t = t0 + i

            @pl.when((i & 1) == 0)
            def _():
                pltpu.make_async_copy(src_hbm.at[t0], ivA, semIA).wait()

                @pl.when(i + 1 < cnt)
                def _():
                    pltpu.async_copy(src_hbm.at[t + 1], ivB, semIB)

                @pl.when(i >= 2)
                def _():
                    pltpu.make_async_copy(slabA.at[pl.ds(0, KW * B)], dst_at(t0), semA).wait()
                build(slabA, ivA, oldA)
                pltpu.async_copy(slabA.at[pl.ds(0, KW * B)], dst_at(t), semA)

            @pl.when((i & 1) == 1)
            def _():
                pltpu.make_async_copy(src_hbm.at[t0], ivB, semIB).wait()

                @pl.when(i + 1 < cnt)
                def _():
                    pltpu.async_copy(src_hbm.at[t + 1], ivA, semIA)

                @pl.when(i >= 2)
                def _():
                    pltpu.make_async_copy(slabB.at[pl.ds(0, KW * B)], dst_at(t0), semB).wait()
                build(slabB, ivB, oldB)
                pltpu.async_copy(slabB.at[pl.ds(0, KW * B)], dst_at(t), semB)

            return c

        lax.fori_loop(0, cnt, step, 0)
        pltpu.make_async_copy(slabA.at[pl.ds(0, KW * B)], dst_at(t0), semA).wait()
        pltpu.make_async_copy(slabB.at[pl.ds(0, KW * B)], dst_at(t0), semB).wait()

    run(rT_hbm, lambda t: r_out.at[t], 0, False)


def kernel(rgap, sgap, pcount, W):
    rT = rgap.T
    sT = sgap.T
    pT = pcount.T
    r_oh = _sc_onehots(rT)
    emb, s_oh, p_oh = _tc_emb(rT, sT, pT, W)
    r_oh = r_oh.reshape(T, NRG, B).transpose(2, 0, 1)
    return (r_oh, s_oh.transpose(2, 0, 1), p_oh.transpose(2, 0, 1),
            emb.transpose(2, 0, 1))
